# baseline (speedup 1.0000x reference)
import functools

import jax
import jax.numpy as jnp
import numpy as np
from jax import lax
from jax.experimental import pallas as pl
from jax.experimental.pallas import tpu as pltpu

PS = 10
ADJ = 5
K = 7
WS = 15
WR = WS // 2
NOFF = WS * WS
H = 130
OFF = 16
BH, BW = 160, 256
BIG = 1e20

_B, _J = np.meshgrid(np.arange(BW), np.arange(BW), indexing="ij")
BND = np.stack([((_B - _J - (dxi - WR) >= -ADJ)
                 & (_B - _J - (dxi - WR) <= PS - 1 - ADJ)).astype(np.float32)
                for dxi in range(WS)])
BND2 = ((_B - _J >= -(PS - 1 - ADJ)) & (_B - _J <= ADJ)).astype(np.float32)


def _roll2(a, sy, sx):
    a = pltpu.roll(a, sy, a.ndim - 2)
    return pltpu.roll(a, sx, a.ndim - 1)


def _box1(p, axis, anchor):
    n = p.shape[axis]
    r = lambda a, k: pltpu.roll(a, (n - k) % n, axis)
    s2 = p + r(p, 1)
    s4 = s2 + r(s2, 2)
    s8 = s4 + r(s4, 4)
    t = s8 + r(s2, 8)
    return pltpu.roll(t, anchor, axis)


def _box(p, anchor):
    return _box1(_box1(p, p.ndim - 2, anchor), p.ndim - 1, anchor)


def _mm(a, b):
    return jax.lax.dot(a, b, precision=jax.lax.Precision.HIGHEST,
                       preferred_element_type=jnp.float32)


def _prep_body(xe_ref, ye_ref, ne_ref, nx_ref, yep_ref):
    ne_ref[...] = _box((ye_ref[...] ** 2).sum(0), ADJ)
    nx_ref[...] = _box((xe_ref[...] ** 2).sum(0), ADJ)
    for dxi in range(WS):
        yep_ref[dxi] = pltpu.roll(ye_ref[...], (BW + dxi - WR) % BW, 2)


def _dist_body(xe_ref, nx_ref, ne_ref, bnd_ref, yep_ref, out_ref):
    dy = pl.program_id(0) - WR
    sy = lax.rem(-dy + BH, BH)
    z = pltpu.roll(xe_ref[...], sy, 1)
    nxy = pltpu.roll(nx_ref[...], sy, 0)
    ne = ne_ref[...]
    ii = lax.broadcasted_iota(jnp.int32, (BH, 128), 0) + dy
    vy = ((ii >= OFF) & (ii < OFF + H)).astype(jnp.float32)[:, :1]
    jj = lax.broadcasted_iota(jnp.int32, (8, BW), 1)
    for dxi in range(WS):
        dx = dxi - WR
        sx = (BW - dx) % BW
        p = (yep_ref[dxi] * z).sum(0)
        q = _box1(p, 0, ADJ)
        cross = _mm(q, bnd_ref[dxi])
        d = ne + pltpu.roll(nxy, sx, 1) - 2.0 * cross
        mj = ((jj + dx >= OFF) & (jj + dx < OFF + H)).astype(jnp.float32)[:1]
        if dx == 0:
            mj = mj * (dy != 0).astype(jnp.float32)
        m = vy * mj
        out_ref[dxi] = d * m + (1.0 - m) * BIG


def _topk_body(d_ref, tau_ref, dmin_ref, invz_ref):
    rows = d_ref.shape[1]
    m = [d_ref[o] for o in range(K)]
    for t in range(K):
        for u in range(t + 1, K):
            lo = jnp.minimum(m[t], m[u])
            m[u] = jnp.maximum(m[t], m[u])
            m[t] = lo
    for o in range(K, NOFF):
        new = d_ref[o]
        for t in range(K):
            lo = jnp.minimum(m[t], new)
            new = jnp.maximum(m[t], new)
            m[t] = lo
    dmin, tau = m[0], m[K - 1]
    z = jnp.zeros((rows, BW), jnp.float32)
    for o in range(NOFF):
        d = d_ref[o]
        z = z + jnp.where(d <= tau, jnp.exp(dmin - d), 0.0)
    ii = pl.program_id(0) * rows + lax.broadcasted_iota(jnp.int32, (rows, BW), 0)
    jj = lax.broadcasted_iota(jnp.int32, (rows, BW), 1)
    in_img = (ii >= OFF) & (ii < OFF + H) & (jj >= OFF) & (jj < OFF + H)
    tau_ref[...] = tau
    dmin_ref[...] = dmin
    invz_ref[...] = jnp.where(in_img, 1.0 / z, 0.0)


def _agg_body(d_ref, x_ref, bnd2_ref, tau_ref, dmin_ref, invz_ref, out_ref):
    dyi = pl.program_id(0)
    dy = dyi - WR
    sy = lax.rem(-dy + BH, BH)
    xy = pltpu.roll(x_ref[...], sy, 1)
    tau, dmin, invz = tau_ref[...], dmin_ref[...], invz_ref[...]
    acc = jnp.zeros(out_ref.shape, jnp.float32)
    for dxi in range(WS):
        dx = dxi - WR
        sx = (BW - dx) % BW
        d = d_ref[dxi]
        w = jnp.where(d <= tau, jnp.exp(dmin - d), 0.0) * invz
        s = _mm(_box1(w, 0, PS - 1 - ADJ), bnd2_ref[...])
        acc = acc + s[None] * pltpu.roll(xy, sx, 2)

    @pl.when(dyi == 0)
    def _():
        out_ref[...] = jnp.zeros_like(out_ref)

    out_ref[...] += acc

    @pl.when(dyi == WS - 1)
    def _():
        ii = lax.broadcasted_iota(jnp.int32, (BH, BW), 0) - OFF
        jj = lax.broadcasted_iota(jnp.int32, (BH, BW), 1) - OFF
        cy = (jnp.minimum(ii + ADJ, H - 1) - jnp.maximum(ii - (PS - 1 - ADJ), 0)
              + 1).clip(1)
        cx = (jnp.minimum(jj + ADJ, H - 1) - jnp.maximum(jj - (PS - 1 - ADJ), 0)
              + 1).clip(1)
        cnt = (cy * cx).astype(jnp.float32)
        out_ref[...] = out_ref[...] / cnt[None] - x_ref[...]


@functools.partial(jax.jit, static_argnames=("interpret",))
def _n3(x, xe, ye, interpret=False):
    emb = lambda a: jnp.pad(a[0], ((0, 0),
                                   (OFF + 1, BH - OFF - 1 - a.shape[-2]),
                                   (OFF + 1, BW - OFF - 1 - a.shape[-1])))
    xb, xeb, yeb = emb(x), emb(xe), emb(ye)

    ec = xe.shape[1]
    ne, nx, yep = pl.pallas_call(
        _prep_body,
        out_shape=[jax.ShapeDtypeStruct((BH, BW), jnp.float32),
                   jax.ShapeDtypeStruct((BH, BW), jnp.float32),
                   jax.ShapeDtypeStruct((WS, ec, BH, BW), jnp.float32)],
        interpret=interpret,
    )(xeb, yeb)

    dists = pl.pallas_call(
        _dist_body,
        grid=(WS,),
        in_specs=[
            pl.BlockSpec((ec, BH, BW), lambda o: (0, 0, 0)),
            pl.BlockSpec((BH, BW), lambda o: (0, 0)),
            pl.BlockSpec((BH, BW), lambda o: (0, 0)),
            pl.BlockSpec((WS, BW, BW), lambda o: (0, 0, 0)),
            pl.BlockSpec((WS, ec, BH, BW), lambda o: (0, 0, 0, 0)),
        ],
        out_specs=pl.BlockSpec((WS, BH, BW), lambda o: (o, 0, 0)),
        out_shape=jax.ShapeDtypeStruct((NOFF, BH, BW), jnp.float32),
        interpret=interpret,
    )(xeb, nx, ne, jnp.asarray(BND), yep)

    rows = 8
    tau, dmin, invz = pl.pallas_call(
        _topk_body,
        grid=(BH // rows,),
        in_specs=[pl.BlockSpec((NOFF, rows, BW), lambda i: (0, i, 0))],
        out_specs=[pl.BlockSpec((rows, BW), lambda i: (i, 0))] * 3,
        out_shape=[jax.ShapeDtypeStruct((BH, BW), jnp.float32)] * 3,
        interpret=interpret,
    )(dists)

    zagg = pl.pallas_call(
        _agg_body,
        grid=(WS,),
        in_specs=[
            pl.BlockSpec((WS, BH, BW), lambda o: (o, 0, 0)),
            pl.BlockSpec((3, BH, BW), lambda o: (0, 0, 0)),
            pl.BlockSpec((BW, BW), lambda o: (0, 0)),
            pl.BlockSpec((BH, BW), lambda o: (0, 0)),
            pl.BlockSpec((BH, BW), lambda o: (0, 0)),
            pl.BlockSpec((BH, BW), lambda o: (0, 0)),
        ],
        out_specs=pl.BlockSpec((3, BH, BW), lambda o: (0, 0, 0)),
        out_shape=jax.ShapeDtypeStruct((3, BH, BW), jnp.float32),
        interpret=interpret,
    )(dists, xb, jnp.asarray(BND2), tau, dmin, invz)

    zc = zagg[:, OFF + 1:OFF + H - 1, OFF + 1:OFF + H - 1]
    return jnp.concatenate([x, zc[None]], axis=1)


def kernel(x, xe, ye):
    return _n3(x, xe, ye)


# speedup vs baseline: 1.0538x; 1.0538x over previous
import functools

import jax
import jax.numpy as jnp
import numpy as np
from jax import lax
from jax.experimental import pallas as pl
from jax.experimental.pallas import tpu as pltpu

PS = 10
ADJ = 5
K = 7
WS = 15
WR = WS // 2
NOFF = WS * WS
H = 130
OFF = 16
BH, BW = 160, 256
BIG = 1e20

_B, _J = np.meshgrid(np.arange(BW), np.arange(BW), indexing="ij")
BND = np.stack([((_B - _J - (dxi - WR) >= -ADJ)
                 & (_B - _J - (dxi - WR) <= PS - 1 - ADJ)).astype(np.float32)
                for dxi in range(WS)])
BND2 = ((_B - _J >= -(PS - 1 - ADJ)) & (_B - _J <= ADJ)).astype(np.float32)


def _roll2(a, sy, sx):
    a = pltpu.roll(a, sy, a.ndim - 2)
    return pltpu.roll(a, sx, a.ndim - 1)


def _box1(p, axis, anchor):
    n = p.shape[axis]
    r = lambda a, k: pltpu.roll(a, (n - k) % n, axis)
    s2 = p + r(p, 1)
    s4 = s2 + r(s2, 2)
    s8 = s4 + r(s4, 4)
    t = s8 + r(s2, 8)
    return pltpu.roll(t, anchor, axis)


def _box(p, anchor):
    return _box1(_box1(p, p.ndim - 2, anchor), p.ndim - 1, anchor)


def _mm(a, b):
    return jax.lax.dot(a, b, precision=jax.lax.Precision.HIGHEST,
                       preferred_element_type=jnp.float32)


def _prep_body(xe_ref, ye_ref, ne_ref, nx_ref):
    ne_ref[...] = _box((ye_ref[...] ** 2).sum(0), ADJ)
    nx_ref[...] = _box((xe_ref[...] ** 2).sum(0), ADJ)


def _dist_body(xe_ref, ye_ref, nx_ref, ne_ref, bnd_ref, out_ref, yep_ref):
    dyi = pl.program_id(0)
    dy = dyi - WR

    @pl.when(dyi == 0)
    def _():
        for dxi in range(WS):
            yep_ref[dxi] = pltpu.roll(ye_ref[...], (BW + dxi - WR) % BW, 2)

    sy = lax.rem(-dy + BH, BH)
    z = pltpu.roll(xe_ref[...], sy, 1)
    nxy = pltpu.roll(nx_ref[...], sy, 0)
    ne = ne_ref[...]
    ii = lax.broadcasted_iota(jnp.int32, (BH, 128), 0) + dy
    vy = ((ii >= OFF) & (ii < OFF + H)).astype(jnp.float32)[:, :1]
    jj = lax.broadcasted_iota(jnp.int32, (8, BW), 1)
    for dxi in range(WS):
        dx = dxi - WR
        sx = (BW - dx) % BW
        p = (yep_ref[dxi] * z).sum(0)
        q = _box1(p, 0, ADJ)
        cross = _mm(q, bnd_ref[dxi])
        d = ne + pltpu.roll(nxy, sx, 1) - 2.0 * cross
        mj = ((jj + dx >= OFF) & (jj + dx < OFF + H)).astype(jnp.float32)[:1]
        if dx == 0:
            mj = mj * (dy != 0).astype(jnp.float32)
        m = vy * mj
        out_ref[dxi] = d * m + (1.0 - m) * BIG


def _topk_body(d_ref, tau_ref, dmin_ref, invz_ref):
    rows = d_ref.shape[1]
    m = [d_ref[o] for o in range(K)]
    for t in range(K):
        for u in range(t + 1, K):
            lo = jnp.minimum(m[t], m[u])
            m[u] = jnp.maximum(m[t], m[u])
            m[t] = lo
    for o in range(K, NOFF):
        new = d_ref[o]
        for t in range(K):
            lo = jnp.minimum(m[t], new)
            new = jnp.maximum(m[t], new)
            m[t] = lo
    dmin, tau = m[0], m[K - 1]
    z = jnp.zeros((rows, BW), jnp.float32)
    for o in range(NOFF):
        d = d_ref[o]
        z = z + jnp.where(d <= tau, jnp.exp(dmin - d), 0.0)
    ii = pl.program_id(0) * rows + lax.broadcasted_iota(jnp.int32, (rows, BW), 0)
    jj = lax.broadcasted_iota(jnp.int32, (rows, BW), 1)
    in_img = (ii >= OFF) & (ii < OFF + H) & (jj >= OFF) & (jj < OFF + H)
    tau_ref[...] = tau
    dmin_ref[...] = dmin
    invz_ref[...] = jnp.where(in_img, 1.0 / z, 0.0)


def _agg_body(d_ref, x_ref, bnd2_ref, tau_ref, dmin_ref, invz_ref, out_ref):
    dyi = pl.program_id(0)
    dy = dyi - WR
    sy = lax.rem(-dy + BH, BH)
    xy = pltpu.roll(x_ref[...], sy, 1)
    tau, dmin, invz = tau_ref[...], dmin_ref[...], invz_ref[...]
    acc = jnp.zeros(out_ref.shape, jnp.float32)
    for dxi in range(WS):
        dx = dxi - WR
        sx = (BW - dx) % BW
        d = d_ref[dxi]
        w = jnp.where(d <= tau, jnp.exp(dmin - d), 0.0) * invz
        s = _mm(_box1(w, 0, PS - 1 - ADJ), bnd2_ref[...])
        acc = acc + s[None] * pltpu.roll(xy, sx, 2)

    @pl.when(dyi == 0)
    def _():
        out_ref[...] = jnp.zeros_like(out_ref)

    out_ref[...] += acc

    @pl.when(dyi == WS - 1)
    def _():
        ii = lax.broadcasted_iota(jnp.int32, (BH, BW), 0) - OFF
        jj = lax.broadcasted_iota(jnp.int32, (BH, BW), 1) - OFF
        cy = (jnp.minimum(ii + ADJ, H - 1) - jnp.maximum(ii - (PS - 1 - ADJ), 0)
              + 1).clip(1)
        cx = (jnp.minimum(jj + ADJ, H - 1) - jnp.maximum(jj - (PS - 1 - ADJ), 0)
              + 1).clip(1)
        cnt = (cy * cx).astype(jnp.float32)
        out_ref[...] = out_ref[...] / cnt[None] - x_ref[...]


@functools.partial(jax.jit, static_argnames=("interpret",))
def _n3(x, xe, ye, interpret=False):
    emb = lambda a: jnp.pad(a[0], ((0, 0),
                                   (OFF + 1, BH - OFF - 1 - a.shape[-2]),
                                   (OFF + 1, BW - OFF - 1 - a.shape[-1])))
    xb, xeb, yeb = emb(x), emb(xe), emb(ye)

    ec = xe.shape[1]
    ne, nx = pl.pallas_call(
        _prep_body,
        out_shape=[jax.ShapeDtypeStruct((BH, BW), jnp.float32),
                   jax.ShapeDtypeStruct((BH, BW), jnp.float32)],
        interpret=interpret,
    )(xeb, yeb)

    dists = pl.pallas_call(
        _dist_body,
        grid=(WS,),
        in_specs=[
            pl.BlockSpec((ec, BH, BW), lambda o: (0, 0, 0)),
            pl.BlockSpec((ec, BH, BW), lambda o: (0, 0, 0)),
            pl.BlockSpec((BH, BW), lambda o: (0, 0)),
            pl.BlockSpec((BH, BW), lambda o: (0, 0)),
            pl.BlockSpec((WS, BW, BW), lambda o: (0, 0, 0)),
        ],
        out_specs=pl.BlockSpec((WS, BH, BW), lambda o: (o, 0, 0)),
        out_shape=jax.ShapeDtypeStruct((NOFF, BH, BW), jnp.float32),
        scratch_shapes=[pltpu.VMEM((WS, ec, BH, BW), jnp.float32)],
        interpret=interpret,
    )(xeb, yeb, nx, ne, jnp.asarray(BND))

    rows = 8
    tau, dmin, invz = pl.pallas_call(
        _topk_body,
        grid=(BH // rows,),
        in_specs=[pl.BlockSpec((NOFF, rows, BW), lambda i: (0, i, 0))],
        out_specs=[pl.BlockSpec((rows, BW), lambda i: (i, 0))] * 3,
        out_shape=[jax.ShapeDtypeStruct((BH, BW), jnp.float32)] * 3,
        interpret=interpret,
    )(dists)

    zagg = pl.pallas_call(
        _agg_body,
        grid=(WS,),
        in_specs=[
            pl.BlockSpec((WS, BH, BW), lambda o: (o, 0, 0)),
            pl.BlockSpec((3, BH, BW), lambda o: (0, 0, 0)),
            pl.BlockSpec((BW, BW), lambda o: (0, 0)),
            pl.BlockSpec((BH, BW), lambda o: (0, 0)),
            pl.BlockSpec((BH, BW), lambda o: (0, 0)),
            pl.BlockSpec((BH, BW), lambda o: (0, 0)),
        ],
        out_specs=pl.BlockSpec((3, BH, BW), lambda o: (0, 0, 0)),
        out_shape=jax.ShapeDtypeStruct((3, BH, BW), jnp.float32),
        interpret=interpret,
    )(dists, xb, jnp.asarray(BND2), tau, dmin, invz)

    zc = zagg[:, OFF + 1:OFF + H - 1, OFF + 1:OFF + H - 1]
    return jnp.concatenate([x, zc[None]], axis=1)


def kernel(x, xe, ye):
    return _n3(x, xe, ye)


# speedup vs baseline: 1.0600x; 1.0059x over previous
import functools

import jax
import jax.numpy as jnp
import numpy as np
from jax import lax
from jax.experimental import pallas as pl
from jax.experimental.pallas import tpu as pltpu

PS = 10
ADJ = 5
K = 7
WS = 15
WR = WS // 2
NOFF = WS * WS
H = 130
OFF = 16
BH, BW = 160, 160
BIG = 1e20

_B, _J = np.meshgrid(np.arange(BW), np.arange(BW), indexing="ij")
BND = np.stack([((_B - _J - (dxi - WR) >= -ADJ)
                 & (_B - _J - (dxi - WR) <= PS - 1 - ADJ)).astype(np.float32)
                for dxi in range(WS)])
BND2 = ((_B - _J >= -(PS - 1 - ADJ)) & (_B - _J <= ADJ)).astype(np.float32)


def _roll2(a, sy, sx):
    a = pltpu.roll(a, sy, a.ndim - 2)
    return pltpu.roll(a, sx, a.ndim - 1)


def _box1(p, axis, anchor):
    n = p.shape[axis]
    r = lambda a, k: pltpu.roll(a, (n - k) % n, axis)
    s2 = p + r(p, 1)
    s4 = s2 + r(s2, 2)
    s8 = s4 + r(s4, 4)
    t = s8 + r(s2, 8)
    return pltpu.roll(t, anchor, axis)


def _box(p, anchor):
    return _box1(_box1(p, p.ndim - 2, anchor), p.ndim - 1, anchor)


def _mm(a, b):
    return jax.lax.dot(a, b, precision=jax.lax.Precision.HIGHEST,
                       preferred_element_type=jnp.float32)


def _prep_body(xe_ref, ye_ref, ne_ref, nx_ref):
    ne_ref[...] = _box((ye_ref[...] ** 2).sum(0), ADJ)
    nx_ref[...] = _box((xe_ref[...] ** 2).sum(0), ADJ)


def _dist_body(xe_ref, ye_ref, nx_ref, ne_ref, bnd_ref, out_ref, yep_ref):
    dyi = pl.program_id(0)
    dy = dyi - WR

    @pl.when(dyi == 0)
    def _():
        for dxi in range(WS):
            yep_ref[dxi] = pltpu.roll(ye_ref[...], (BW + dxi - WR) % BW, 2)

    sy = lax.rem(-dy + BH, BH)
    z = pltpu.roll(xe_ref[...], sy, 1)
    nxy = pltpu.roll(nx_ref[...], sy, 0)
    ne = ne_ref[...]
    ii = lax.broadcasted_iota(jnp.int32, (BH, 128), 0) + dy
    vy = ((ii >= OFF) & (ii < OFF + H)).astype(jnp.float32)[:, :1]
    jj = lax.broadcasted_iota(jnp.int32, (8, BW), 1)
    for dxi in range(WS):
        dx = dxi - WR
        sx = (BW - dx) % BW
        p = (yep_ref[dxi] * z).sum(0)
        q = _box1(p, 0, ADJ)
        cross = _mm(q, bnd_ref[dxi])
        d = ne + pltpu.roll(nxy, sx, 1) - 2.0 * cross
        mj = ((jj + dx >= OFF) & (jj + dx < OFF + H)).astype(jnp.float32)[:1]
        if dx == 0:
            mj = mj * (dy != 0).astype(jnp.float32)
        m = vy * mj
        out_ref[dxi] = d * m + (1.0 - m) * BIG


def _topk_body(d_ref, tau_ref, dmin_ref, invz_ref):
    rows = d_ref.shape[1]
    m = [d_ref[o] for o in range(K)]
    for t in range(K):
        for u in range(t + 1, K):
            lo = jnp.minimum(m[t], m[u])
            m[u] = jnp.maximum(m[t], m[u])
            m[t] = lo
    for o in range(K, NOFF):
        new = d_ref[o]
        for t in range(K):
            lo = jnp.minimum(m[t], new)
            new = jnp.maximum(m[t], new)
            m[t] = lo
    dmin, tau = m[0], m[K - 1]
    z = jnp.zeros((rows, BW), jnp.float32)
    for o in range(NOFF):
        d = d_ref[o]
        z = z + jnp.where(d <= tau, jnp.exp(dmin - d), 0.0)
    ii = pl.program_id(0) * rows + lax.broadcasted_iota(jnp.int32, (rows, BW), 0)
    jj = lax.broadcasted_iota(jnp.int32, (rows, BW), 1)
    in_img = (ii >= OFF) & (ii < OFF + H) & (jj >= OFF) & (jj < OFF + H)
    tau_ref[...] = tau
    dmin_ref[...] = dmin
    invz_ref[...] = jnp.where(in_img, 1.0 / z, 0.0)


def _agg_body(d_ref, x_ref, bnd2_ref, tau_ref, dmin_ref, invz_ref, out_ref):
    dyi = pl.program_id(0)
    dy = dyi - WR
    sy = lax.rem(-dy + BH, BH)
    xy = pltpu.roll(x_ref[...], sy, 1)
    tau, dmin, invz = tau_ref[...], dmin_ref[...], invz_ref[...]
    acc = jnp.zeros(out_ref.shape, jnp.float32)
    for dxi in range(WS):
        dx = dxi - WR
        sx = (BW - dx) % BW
        d = d_ref[dxi]
        w = jnp.where(d <= tau, jnp.exp(dmin - d), 0.0) * invz
        s = _mm(_box1(w, 0, PS - 1 - ADJ), bnd2_ref[...])
        acc = acc + s[None] * pltpu.roll(xy, sx, 2)

    @pl.when(dyi == 0)
    def _():
        out_ref[...] = jnp.zeros_like(out_ref)

    out_ref[...] += acc

    @pl.when(dyi == WS - 1)
    def _():
        ii = lax.broadcasted_iota(jnp.int32, (BH, BW), 0) - OFF
        jj = lax.broadcasted_iota(jnp.int32, (BH, BW), 1) - OFF
        cy = (jnp.minimum(ii + ADJ, H - 1) - jnp.maximum(ii - (PS - 1 - ADJ), 0)
              + 1).clip(1)
        cx = (jnp.minimum(jj + ADJ, H - 1) - jnp.maximum(jj - (PS - 1 - ADJ), 0)
              + 1).clip(1)
        cnt = (cy * cx).astype(jnp.float32)
        out_ref[...] = out_ref[...] / cnt[None] - x_ref[...]


@functools.partial(jax.jit, static_argnames=("interpret",))
def _n3(x, xe, ye, interpret=False):
    emb = lambda a: jnp.pad(a[0], ((0, 0),
                                   (OFF + 1, BH - OFF - 1 - a.shape[-2]),
                                   (OFF + 1, BW - OFF - 1 - a.shape[-1])))
    xb, xeb, yeb = emb(x), emb(xe), emb(ye)

    ec = xe.shape[1]
    ne, nx = pl.pallas_call(
        _prep_body,
        out_shape=[jax.ShapeDtypeStruct((BH, BW), jnp.float32),
                   jax.ShapeDtypeStruct((BH, BW), jnp.float32)],
        interpret=interpret,
    )(xeb, yeb)

    dists = pl.pallas_call(
        _dist_body,
        grid=(WS,),
        in_specs=[
            pl.BlockSpec((ec, BH, BW), lambda o: (0, 0, 0)),
            pl.BlockSpec((ec, BH, BW), lambda o: (0, 0, 0)),
            pl.BlockSpec((BH, BW), lambda o: (0, 0)),
            pl.BlockSpec((BH, BW), lambda o: (0, 0)),
            pl.BlockSpec((WS, BW, BW), lambda o: (0, 0, 0)),
        ],
        out_specs=pl.BlockSpec((WS, BH, BW), lambda o: (o, 0, 0)),
        out_shape=jax.ShapeDtypeStruct((NOFF, BH, BW), jnp.float32),
        scratch_shapes=[pltpu.VMEM((WS, ec, BH, BW), jnp.float32)],
        interpret=interpret,
    )(xeb, yeb, nx, ne, jnp.asarray(BND))

    rows = 8
    tau, dmin, invz = pl.pallas_call(
        _topk_body,
        grid=(BH // rows,),
        in_specs=[pl.BlockSpec((NOFF, rows, BW), lambda i: (0, i, 0))],
        out_specs=[pl.BlockSpec((rows, BW), lambda i: (i, 0))] * 3,
        out_shape=[jax.ShapeDtypeStruct((BH, BW), jnp.float32)] * 3,
        interpret=interpret,
    )(dists)

    zagg = pl.pallas_call(
        _agg_body,
        grid=(WS,),
        in_specs=[
            pl.BlockSpec((WS, BH, BW), lambda o: (o, 0, 0)),
            pl.BlockSpec((3, BH, BW), lambda o: (0, 0, 0)),
            pl.BlockSpec((BW, BW), lambda o: (0, 0)),
            pl.BlockSpec((BH, BW), lambda o: (0, 0)),
            pl.BlockSpec((BH, BW), lambda o: (0, 0)),
            pl.BlockSpec((BH, BW), lambda o: (0, 0)),
        ],
        out_specs=pl.BlockSpec((3, BH, BW), lambda o: (0, 0, 0)),
        out_shape=jax.ShapeDtypeStruct((3, BH, BW), jnp.float32),
        interpret=interpret,
    )(dists, xb, jnp.asarray(BND2), tau, dmin, invz)

    zc = zagg[:, OFF + 1:OFF + H - 1, OFF + 1:OFF + H - 1]
    return jnp.concatenate([x, zc[None]], axis=1)


def kernel(x, xe, ye):
    return _n3(x, xe, ye)


# speedup vs baseline: 1.3762x; 1.2983x over previous
import functools

import jax
import jax.numpy as jnp
import numpy as np
from jax import lax
from jax.experimental import pallas as pl
from jax.experimental.pallas import tpu as pltpu

PS = 10
ADJ = 5
K = 7
WS = 15
WR = WS // 2
NOFF = WS * WS
H = 130
OFF = 16
BH, BW = 160, 160
BIG = 1e20

_B, _J = np.meshgrid(np.arange(BW), np.arange(BW), indexing="ij")
BND = np.stack([((_B - _J - (dxi - WR) >= -ADJ)
                 & (_B - _J - (dxi - WR) <= PS - 1 - ADJ)).astype(np.float32)
                for dxi in range(WS)])
BND2 = ((_B - _J >= -(PS - 1 - ADJ)) & (_B - _J <= ADJ)).astype(np.float32)


def _roll2(a, sy, sx):
    a = pltpu.roll(a, sy, a.ndim - 2)
    return pltpu.roll(a, sx, a.ndim - 1)


def _box1(p, axis, anchor):
    n = p.shape[axis]
    r = lambda a, k: pltpu.roll(a, (n - k) % n, axis)
    s2 = p + r(p, 1)
    s4 = s2 + r(s2, 2)
    s8 = s4 + r(s4, 4)
    t = s8 + r(s2, 8)
    return pltpu.roll(t, anchor, axis)


def _box(p, anchor):
    return _box1(_box1(p, p.ndim - 2, anchor), p.ndim - 1, anchor)


def _mm(a, b):
    ah = a.astype(jnp.bfloat16)
    al = (a - ah.astype(jnp.float32)).astype(jnp.bfloat16)
    return (jax.lax.dot(ah, b, preferred_element_type=jnp.float32)
            + jax.lax.dot(al, b, preferred_element_type=jnp.float32))


def _dist_body(xe_ref, ye_ref, bnd_ref, out_ref, yep_ref, ne_ref, nx_ref):
    dyi = pl.program_id(0)
    dy = dyi - WR

    @pl.when(dyi == 0)
    def _():
        for dxi in range(WS):
            yep_ref[dxi] = pltpu.roll(ye_ref[...], (BW + dxi - WR) % BW, 2)
        ne_ref[...] = _box((ye_ref[...] ** 2).sum(0), ADJ)
        nx_ref[...] = _box((xe_ref[...] ** 2).sum(0), ADJ)

    sy = lax.rem(-dy + BH, BH)
    z = pltpu.roll(xe_ref[...], sy, 1)
    nxy = pltpu.roll(nx_ref[...], sy, 0)
    ne = ne_ref[...]
    ii = lax.broadcasted_iota(jnp.int32, (BH, 128), 0) + dy
    vy = ((ii >= OFF) & (ii < OFF + H)).astype(jnp.float32)[:, :1]
    jj = lax.broadcasted_iota(jnp.int32, (8, BW), 1)
    for dxi in range(WS):
        dx = dxi - WR
        sx = (BW - dx) % BW
        p = (yep_ref[dxi] * z).sum(0)
        q = _box1(p, 0, ADJ)
        cross = _mm(q, bnd_ref[dxi])
        d = ne + pltpu.roll(nxy, sx, 1) - 2.0 * cross
        mj = ((jj + dx >= OFF) & (jj + dx < OFF + H)).astype(jnp.float32)[:1]
        if dx == 0:
            mj = mj * (dy != 0).astype(jnp.float32)
        m = vy * mj
        out_ref[dxi] = d * m + (1.0 - m) * BIG


def _topk_body(d_ref, tau_ref, dmin_ref, invz_ref):
    rows = d_ref.shape[1]
    m = [d_ref[o] for o in range(K)]
    for t in range(K):
        for u in range(t + 1, K):
            lo = jnp.minimum(m[t], m[u])
            m[u] = jnp.maximum(m[t], m[u])
            m[t] = lo
    for o in range(K, NOFF):
        new = d_ref[o]
        for t in range(K):
            lo = jnp.minimum(m[t], new)
            new = jnp.maximum(m[t], new)
            m[t] = lo
    dmin, tau = m[0], m[K - 1]
    z = jnp.zeros((rows, BW), jnp.float32)
    for o in range(NOFF):
        d = d_ref[o]
        z = z + jnp.where(d <= tau, jnp.exp(dmin - d), 0.0)
    ii = pl.program_id(0) * rows + lax.broadcasted_iota(jnp.int32, (rows, BW), 0)
    jj = lax.broadcasted_iota(jnp.int32, (rows, BW), 1)
    in_img = (ii >= OFF) & (ii < OFF + H) & (jj >= OFF) & (jj < OFF + H)
    tau_ref[...] = tau
    dmin_ref[...] = dmin
    invz_ref[...] = jnp.where(in_img, 1.0 / z, 0.0)


def _agg_body(d_ref, x_ref, bnd2_ref, tau_ref, dmin_ref, invz_ref, out_ref):
    dyi = pl.program_id(0)
    dy = dyi - WR
    sy = lax.rem(-dy + BH, BH)
    xy = pltpu.roll(x_ref[...], sy, 1)
    tau, dmin, invz = tau_ref[...], dmin_ref[...], invz_ref[...]
    acc = jnp.zeros(out_ref.shape, jnp.float32)
    for dxi in range(WS):
        dx = dxi - WR
        sx = (BW - dx) % BW
        d = d_ref[dxi]
        w = jnp.where(d <= tau, jnp.exp(dmin - d), 0.0) * invz
        s = _mm(_box1(w, 0, PS - 1 - ADJ), bnd2_ref[...])
        acc = acc + s[None] * pltpu.roll(xy, sx, 2)

    @pl.when(dyi == 0)
    def _():
        out_ref[...] = jnp.zeros_like(out_ref)

    out_ref[...] += acc

    @pl.when(dyi == WS - 1)
    def _():
        ii = lax.broadcasted_iota(jnp.int32, (BH, BW), 0) - OFF
        jj = lax.broadcasted_iota(jnp.int32, (BH, BW), 1) - OFF
        cy = (jnp.minimum(ii + ADJ, H - 1) - jnp.maximum(ii - (PS - 1 - ADJ), 0)
              + 1).clip(1)
        cx = (jnp.minimum(jj + ADJ, H - 1) - jnp.maximum(jj - (PS - 1 - ADJ), 0)
              + 1).clip(1)
        cnt = (cy * cx).astype(jnp.float32)
        out_ref[...] = out_ref[...] / cnt[None] - x_ref[...]


@functools.partial(jax.jit, static_argnames=("interpret",))
def _n3(x, xe, ye, interpret=False):
    emb = lambda a: jnp.pad(a[0], ((0, 0),
                                   (OFF + 1, BH - OFF - 1 - a.shape[-2]),
                                   (OFF + 1, BW - OFF - 1 - a.shape[-1])))
    xb, xeb, yeb = emb(x), emb(xe), emb(ye)

    ec = xe.shape[1]
    dists = pl.pallas_call(
        _dist_body,
        grid=(WS,),
        in_specs=[
            pl.BlockSpec((ec, BH, BW), lambda o: (0, 0, 0)),
            pl.BlockSpec((ec, BH, BW), lambda o: (0, 0, 0)),
            pl.BlockSpec((WS, BW, BW), lambda o: (0, 0, 0)),
        ],
        out_specs=pl.BlockSpec((WS, BH, BW), lambda o: (o, 0, 0)),
        out_shape=jax.ShapeDtypeStruct((NOFF, BH, BW), jnp.float32),
        scratch_shapes=[pltpu.VMEM((WS, ec, BH, BW), jnp.float32),
                        pltpu.VMEM((BH, BW), jnp.float32),
                        pltpu.VMEM((BH, BW), jnp.float32)],
        interpret=interpret,
    )(xeb, yeb, jnp.asarray(BND, dtype=jnp.bfloat16))

    rows = 16
    tau, dmin, invz = pl.pallas_call(
        _topk_body,
        grid=(BH // rows,),
        in_specs=[pl.BlockSpec((NOFF, rows, BW), lambda i: (0, i, 0))],
        out_specs=[pl.BlockSpec((rows, BW), lambda i: (i, 0))] * 3,
        out_shape=[jax.ShapeDtypeStruct((BH, BW), jnp.float32)] * 3,
        interpret=interpret,
    )(dists)

    zagg = pl.pallas_call(
        _agg_body,
        grid=(WS,),
        in_specs=[
            pl.BlockSpec((WS, BH, BW), lambda o: (o, 0, 0)),
            pl.BlockSpec((3, BH, BW), lambda o: (0, 0, 0)),
            pl.BlockSpec((BW, BW), lambda o: (0, 0)),
            pl.BlockSpec((BH, BW), lambda o: (0, 0)),
            pl.BlockSpec((BH, BW), lambda o: (0, 0)),
            pl.BlockSpec((BH, BW), lambda o: (0, 0)),
        ],
        out_specs=pl.BlockSpec((3, BH, BW), lambda o: (0, 0, 0)),
        out_shape=jax.ShapeDtypeStruct((3, BH, BW), jnp.float32),
        interpret=interpret,
    )(dists, xb, jnp.asarray(BND2, dtype=jnp.bfloat16), tau, dmin, invz)

    zc = zagg[:, OFF + 1:OFF + H - 1, OFF + 1:OFF + H - 1]
    return jnp.concatenate([x, zc[None]], axis=1)


def kernel(x, xe, ye):
    return _n3(x, xe, ye)
